# baseline (speedup 1.0000x reference)
import numpy as np

import jax
import jax.numpy as jnp
from jax import lax
from jax.experimental import pallas as pl
from jax.experimental.pallas import tpu as pltpu
from jax.experimental.pallas import tpu_sc as plsc

_FEATURE_DIMS = [119, 5, 12, 12, 10, 6, 6, 2, 2]
_NF = len(_FEATURE_DIMS)
_C = 256
_K = sum(_FEATURE_DIMS)
_NB = 16
_NR = _NB * _NF
_NW = 32
_NSB = 100000 // _NB
_PER = _NSB // _NW
_XW = _NSB - _PER * _NW


def _compute(rows_ref, out_ref):
    @plsc.parallel_loop(0, _NB, 1, unroll=2)
    def node_body(j):
        for c in range(_C // 16):
            sl = pl.ds(c * 16, 16)
            acc = rows_ref[j * _NF, sl]
            for i in range(1, _NF):
                acc = acc + rows_ref[j * _NF + i, sl]
            out_ref[j, sl] = acc


def _sc_body(table_hbm, idx_hbm, out_hbm, idx_v, rows_v, out_v, sem0, sem1):
    wid = lax.axis_index("s") * 2 + lax.axis_index("c")
    n_sb = jnp.where(wid < _XW, _PER + 1, _PER)
    sems = (sem0, sem1)

    def start_gather(m, b):
        pltpu.sync_copy(idx_hbm.at[m], idx_v.at[b])
        for h in range(2):
            pltpu.async_copy(
                table_hbm.at[idx_v.at[b, h]],
                rows_v.at[b, pl.ds(h * (_NR // 2), _NR // 2), :], sems[b])

    def finish(m, b):
        for h in range(2):
            pltpu.make_async_copy(
                table_hbm.at[idx_v.at[b, h]],
                rows_v.at[b, pl.ds(h * (_NR // 2), _NR // 2), :],
                sems[b]).wait()
        _compute(rows_v.at[b], out_v.at[b])
        pltpu.sync_copy(out_v.at[b], out_hbm.at[pl.ds(m * _NB, _NB), :])

    start_gather(wid, 0)
    start_gather(wid + _NW, 1)

    def phase_body(p, carry):
        k0 = 2 * p
        k1 = 2 * p + 1

        finish(wid + k0 * _NW, 0)

        @pl.when(k0 + 2 < n_sb)
        def _():
            start_gather(wid + (k0 + 2) * _NW, 0)

        @pl.when(k1 < n_sb)
        def _():
            finish(wid + k1 * _NW, 1)

            @pl.when(k1 + 2 < n_sb)
            def _():
                start_gather(wid + (k1 + 2) * _NW, 1)

        return carry

    lax.fori_loop(0, (_PER + 2) // 2, phase_body, 0)


def kernel(x, batch, emb_0, emb_1, emb_2, emb_3, emb_4, emb_5, emb_6, emb_7,
           emb_8):
    del batch
    embs = [emb_0, emb_1, emb_2, emb_3, emb_4, emb_5, emb_6, emb_7, emb_8]
    table = jnp.concatenate(embs, axis=0)

    offs = np.zeros((1, _NF), np.int32)
    acc = 0
    for i, d in enumerate(_FEATURE_DIMS):
        offs[0, i] = acc
        acc += d
    n = x.shape[0]
    flat = x.astype(jnp.int32) + jnp.asarray(offs)
    idx = flat.reshape(_NSB, 2, _NR // 2)

    mesh = plsc.VectorSubcoreMesh(
        core_axis_name="c", subcore_axis_name="s", num_cores=2,
        num_subcores=16)
    k = pl.kernel(
        _sc_body,
        out_type=jax.ShapeDtypeStruct((n, _C), jnp.float32),
        mesh=mesh,
        scratch_types=[
            pltpu.VMEM((2, 2, _NR // 2), jnp.int32),
            pltpu.VMEM((2, _NR, _C), jnp.float32),
            pltpu.VMEM((2, _NB, _C), jnp.float32),
            pltpu.SemaphoreType.DMA,
            pltpu.SemaphoreType.DMA,
        ],
        compiler_params=pltpu.CompilerParams(needs_layout_passes=False),
    )
    return k(table, idx)


# speedup vs baseline: 1.1408x; 1.1408x over previous
import functools

import numpy as np

import jax
import jax.numpy as jnp
from jax import lax
from jax.experimental import pallas as pl
from jax.experimental.pallas import tpu as pltpu
from jax.experimental.pallas import tpu_sc as plsc

_FEATURE_DIMS = [119, 5, 12, 12, 10, 6, 6, 2, 2]
_NF = len(_FEATURE_DIMS)
_C = 256
_K = sum(_FEATURE_DIMS)
_K_PAD = 176
_NB = 32
_NW = 32
_NSB = 100000 // _NB
_REM = _NSB - (_NSB // _NW) * _NW


def _sc_body(table_hbm, idx_hbm, out_hbm, table_v, idx_v, out_v):
    wid = lax.axis_index("s") * 2 + lax.axis_index("c")
    pltpu.sync_copy(table_hbm, table_v)
    n_sb = jnp.where(wid < _REM, _NSB // _NW + 1, _NSB // _NW)

    def sb_body(k, carry):
        sb = wid + k * _NW
        iota = lax.broadcasted_iota(jnp.int32, (16,), 0)
        pltpu.sync_copy(idx_hbm.at[sb], idx_v)
        for g in range(_NB // 16):
            rows = [idx_v[i, pl.ds(g * 16, 16)] for i in range(_NF)]
            base = [r * _C for r in rows]
            dst_rows = iota + (g * 16)

            @plsc.parallel_loop(0, _C, 1, unroll=4)
            def ch_body(ch, base=base, dst_rows=dst_rows):
                gs = [plsc.load_gather(table_v, [base[i] + ch])
                      for i in range(_NF)]
                while len(gs) > 1:
                    gs = [a + b for a, b in zip(gs[::2], gs[1::2])] + (
                        [gs[-1]] if len(gs) % 2 else [])
                col = jnp.full((16,), 0, jnp.int32) + ch
                plsc.store_scatter(out_v, [dst_rows, col], gs[0])
        pltpu.sync_copy(out_v, out_hbm.at[pl.ds(sb * _NB, _NB), :])
        return carry

    lax.fori_loop(0, n_sb, sb_body, 0)


def kernel(x, batch, emb_0, emb_1, emb_2, emb_3, emb_4, emb_5, emb_6, emb_7,
           emb_8):
    del batch
    embs = [emb_0, emb_1, emb_2, emb_3, emb_4, emb_5, emb_6, emb_7, emb_8]
    table = jnp.concatenate(embs, axis=0)
    table = jnp.pad(table, ((0, _K_PAD - _K), (0, 0))).reshape(-1)

    offs = np.zeros((1, _NF), np.int32)
    acc = 0
    for i, d in enumerate(_FEATURE_DIMS):
        offs[0, i] = acc
        acc += d
    n = x.shape[0]
    flat = x.astype(jnp.int32) + jnp.asarray(offs)
    idx = jnp.transpose(flat.reshape(_NSB, _NB, _NF), (0, 2, 1))
    idx = jnp.concatenate(
        [idx, jnp.zeros((_NSB, 16 - _NF, _NB), jnp.int32)], axis=1)

    mesh = plsc.VectorSubcoreMesh(
        core_axis_name="c", subcore_axis_name="s", num_cores=2,
        num_subcores=16)
    k = pl.kernel(
        _sc_body,
        out_type=jax.ShapeDtypeStruct((n, _C), jnp.float32),
        mesh=mesh,
        scratch_types=[
            pltpu.VMEM((_K_PAD * _C,), jnp.float32),
            pltpu.VMEM((16, _NB), jnp.int32),
            pltpu.VMEM((_NB, _C), jnp.float32),
        ],
        compiler_params=pltpu.CompilerParams(needs_layout_passes=False),
    )
    return k(table, idx)


# speedup vs baseline: 5.1426x; 4.5077x over previous
import functools

import numpy as np

import jax
import jax.numpy as jnp
from jax import lax
from jax.experimental import pallas as pl
from jax.experimental.pallas import tpu as pltpu
from jax.experimental.pallas import tpu_sc as plsc

_FEATURE_DIMS = [119, 5, 12, 12, 10, 6, 6, 2, 2]
_NF = len(_FEATURE_DIMS)
_C = 256
_K = sum(_FEATURE_DIMS)
_K_PAD = 176
_STRIDE = 257
_NB = 32
_NW = 32
_NSB = 100000 // _NB
_REM = _NSB - (_NSB // _NW) * _NW


def _sc_body(table_hbm, idx_hbm, out_hbm, table_v, idx_v, out_v):
    wid = lax.axis_index("s") * 2 + lax.axis_index("c")
    pltpu.sync_copy(table_hbm, table_v)
    n_sb = jnp.where(wid < _REM, _NSB // _NW + 1, _NSB // _NW)

    def sb_body(k, carry):
        sb = wid + k * _NW
        iota = lax.broadcasted_iota(jnp.int32, (16,), 0)
        pltpu.sync_copy(idx_hbm.at[sb], idx_v)
        for g in range(_NB // 16):
            rows = [idx_v[i, pl.ds(g * 16, 16)] for i in range(_NF)]
            base = [r * _STRIDE for r in rows]
            dst_rows = iota + (g * 16)

            @plsc.parallel_loop(0, _C, 1, unroll=4)
            def ch_body(ch, base=base, dst_rows=dst_rows):
                gs = [plsc.load_gather(table_v, [base[i] + ch])
                      for i in range(_NF)]
                while len(gs) > 1:
                    gs = [a + b for a, b in zip(gs[::2], gs[1::2])] + (
                        [gs[-1]] if len(gs) % 2 else [])
                col = jnp.full((16,), 0, jnp.int32) + ch
                plsc.store_scatter(out_v, [dst_rows, col], gs[0])
        pltpu.sync_copy(out_v, out_hbm.at[pl.ds(sb * _NB, _NB), :])
        return carry

    lax.fori_loop(0, n_sb, sb_body, 0)


def kernel(x, batch, emb_0, emb_1, emb_2, emb_3, emb_4, emb_5, emb_6, emb_7,
           emb_8):
    del batch
    embs = [emb_0, emb_1, emb_2, emb_3, emb_4, emb_5, emb_6, emb_7, emb_8]
    table = jnp.concatenate(embs, axis=0)
    table = jnp.pad(table, ((0, _K_PAD - _K), (0, _STRIDE - _C))).reshape(-1)

    offs = np.zeros((1, _NF), np.int32)
    acc = 0
    for i, d in enumerate(_FEATURE_DIMS):
        offs[0, i] = acc
        acc += d
    n = x.shape[0]
    flat = x.astype(jnp.int32) + jnp.asarray(offs)
    idx = jnp.transpose(flat.reshape(_NSB, _NB, _NF), (0, 2, 1))
    idx = jnp.concatenate(
        [idx, jnp.zeros((_NSB, 16 - _NF, _NB), jnp.int32)], axis=1)

    mesh = plsc.VectorSubcoreMesh(
        core_axis_name="c", subcore_axis_name="s", num_cores=2,
        num_subcores=16)
    k = pl.kernel(
        _sc_body,
        out_type=jax.ShapeDtypeStruct((n, _C), jnp.float32),
        mesh=mesh,
        scratch_types=[
            pltpu.VMEM((_K_PAD * _STRIDE,), jnp.float32),
            pltpu.VMEM((16, _NB), jnp.int32),
            pltpu.VMEM((_NB, _C), jnp.float32),
        ],
        compiler_params=pltpu.CompilerParams(needs_layout_passes=False),
    )
    return k(table, idx)


# speedup vs baseline: 6.3921x; 1.2430x over previous
import functools

import numpy as np

import jax
import jax.numpy as jnp
from jax import lax
from jax.experimental import pallas as pl
from jax.experimental.pallas import tpu as pltpu
from jax.experimental.pallas import tpu_sc as plsc

_FEATURE_DIMS = [119, 5, 12, 12, 10, 6, 6, 2, 2]
_NF = len(_FEATURE_DIMS)
_C = 256
_K = sum(_FEATURE_DIMS)
_K_PAD = 176
_STRIDE = 257
_NB = 32
_NW = 32
_NSB = 100000 // _NB
_REM = _NSB - (_NSB // _NW) * _NW


def _sc_body(table_hbm, idx_hbm, out_hbm, table_v, idx_v, out_v):
    wid = lax.axis_index("s") * 2 + lax.axis_index("c")
    pltpu.sync_copy(table_hbm, table_v)
    n_sb = jnp.where(wid < _REM, _NSB // _NW + 1, _NSB // _NW)

    def sb_body(k, carry):
        sb = wid + k * _NW
        iota = lax.broadcasted_iota(jnp.int32, (16,), 0)
        pltpu.sync_copy(idx_hbm.at[sb], idx_v)
        for g in range(_NB // 16):
            rows = [idx_v[i, pl.ds(g * 16, 16)] for i in range(_NF)]
            base = [r * _STRIDE for r in rows]
            dst_rows = iota + (g * 16)

            @plsc.parallel_loop(0, _C, 1, unroll=4)
            def ch_body(ch, base=base, dst_rows=dst_rows, iota=iota):
                colv = (ch & -16) + ((iota + ch) & 15)
                gs = [plsc.load_gather(table_v, [base[i] + colv])
                      for i in range(_NF)]
                while len(gs) > 1:
                    gs = [a + b for a, b in zip(gs[::2], gs[1::2])] + (
                        [gs[-1]] if len(gs) % 2 else [])
                plsc.store_scatter(out_v, [dst_rows, colv], gs[0])
        pltpu.sync_copy(out_v, out_hbm.at[pl.ds(sb * _NB, _NB), :])
        return carry

    lax.fori_loop(0, n_sb, sb_body, 0)


def kernel(x, batch, emb_0, emb_1, emb_2, emb_3, emb_4, emb_5, emb_6, emb_7,
           emb_8):
    del batch
    embs = [emb_0, emb_1, emb_2, emb_3, emb_4, emb_5, emb_6, emb_7, emb_8]
    table = jnp.concatenate(embs, axis=0)
    table = jnp.pad(table, ((0, _K_PAD - _K), (0, _STRIDE - _C))).reshape(-1)

    offs = np.zeros((1, _NF), np.int32)
    acc = 0
    for i, d in enumerate(_FEATURE_DIMS):
        offs[0, i] = acc
        acc += d
    n = x.shape[0]
    flat = x.astype(jnp.int32) + jnp.asarray(offs)
    idx = jnp.transpose(flat.reshape(_NSB, _NB, _NF), (0, 2, 1))
    idx = jnp.concatenate(
        [idx, jnp.zeros((_NSB, 16 - _NF, _NB), jnp.int32)], axis=1)

    mesh = plsc.VectorSubcoreMesh(
        core_axis_name="c", subcore_axis_name="s", num_cores=2,
        num_subcores=16)
    k = pl.kernel(
        _sc_body,
        out_type=jax.ShapeDtypeStruct((n, _C), jnp.float32),
        mesh=mesh,
        scratch_types=[
            pltpu.VMEM((_K_PAD * _STRIDE,), jnp.float32),
            pltpu.VMEM((16, _NB), jnp.int32),
            pltpu.VMEM((_NB, _C), jnp.float32),
        ],
        compiler_params=pltpu.CompilerParams(needs_layout_passes=False),
    )
    return k(table, idx)
